# baseline (speedup 1.0000x reference)
import jax
import jax.numpy as jnp
from jax import lax
from jax.experimental import pallas as pl
from jax.experimental.pallas import tpu as pltpu, tpu_sc as plsc

_ROWS = 8192
_D = 1024
_NC = 2
_NS = 16
_NW = _NC * _NS
_RPW = _ROWS // _NW
_C = 32
_SCH = _C
_NBUF = 2
_SPC = 32
_NCH = 6


def _copy_body(W_hbm, out_hbm, buf, spbuf, lsem, ssem, l2sem, s2sem):
    wid = lax.axis_index("s") * _NC + lax.axis_index("c")
    base = wid * _RPW

    def load(i, b):
        return pltpu.make_async_copy(
            W_hbm.at[pl.ds(base + i * _C, _C)], buf.at[b], lsem.at[b])

    def store(i, b):
        return pltpu.make_async_copy(
            buf.at[b], out_hbm.at[pl.ds(base + i * _C, _C)], ssem.at[b])

    sid = lax.axis_index("s")
    spbase = base + _SCH * _NCH

    def sload(j, b):
        return pltpu.make_async_copy(
            W_hbm.at[pl.ds(spbase + j * _SPC, _SPC)], spbuf.at[sid, b],
            l2sem.at[b])

    def sstore(j, b):
        return pltpu.make_async_copy(
            spbuf.at[sid, b], out_hbm.at[pl.ds(spbase + j * _SPC, _SPC)],
            s2sem.at[b])

    sload(0, 0).start()
    sload(1, 1).start()
    load(0, 0).start()
    for i in range(_NCH):
        b = i % _NBUF
        if i + 1 < _NCH:
            nb = (i + 1) % _NBUF
            if i + 1 >= _NBUF:
                store(i + 1 - _NBUF, nb).wait()
            load(i + 1, nb).start()
        load(i, b).wait()
        store(i, b).start()
        if i == 2:
            sload(0, 0).wait()
            sstore(0, 0).start()
        if i == 4:
            sload(1, 1).wait()
            sstore(1, 1).start()
    for i in range(max(0, _NCH - _NBUF), _NCH):
        store(i, i % _NBUF).wait()
    sstore(0, 0).wait()
    sstore(1, 1).wait()


@jax.jit
def _copy(W):
    mesh = plsc.VectorSubcoreMesh(core_axis_name="c", subcore_axis_name="s")
    return pl.kernel(
        _copy_body,
        out_type=jax.ShapeDtypeStruct((_ROWS, _D), jnp.float32),
        mesh=mesh,
        scratch_types=[
            pltpu.VMEM((_NBUF, _C, _D), jnp.float32),
            pltpu.VMEM_SHARED((_NS, 2, _SPC, _D), jnp.float32),
            pltpu.SemaphoreType.DMA((_NBUF,)),
            pltpu.SemaphoreType.DMA((_NBUF,)),
            pltpu.SemaphoreType.DMA((2,)),
            pltpu.SemaphoreType.DMA((2,)),
        ],
    )(W)


def kernel(x, W):
    del x
    return _copy(W)[None]


# speedup vs baseline: 1.0019x; 1.0019x over previous
import jax
import jax.numpy as jnp
from jax import lax
from jax.experimental import pallas as pl
from jax.experimental.pallas import tpu as pltpu, tpu_sc as plsc

_ROWS = 8192
_D = 1024
_NC = 2
_NS = 16
_NW = _NC * _NS
_RPW = _ROWS // _NW
_C = 32
_NBUF = 2
_NCH = _RPW // _C


def _copy_body(W_hbm, out_hbm, buf, lsem, ssem):
    wid = lax.axis_index("s") * _NC + lax.axis_index("c")
    base = wid * _RPW

    def load(i, b):
        return pltpu.make_async_copy(
            W_hbm.at[pl.ds(base + i * _C, _C)], buf.at[b], lsem.at[b])

    def store(i, b):
        return pltpu.make_async_copy(
            buf.at[b], out_hbm.at[pl.ds(base + i * _C, _C)], ssem.at[b])

    load(0, 0).start()
    for i in range(_NCH):
        b = i % _NBUF
        if i + 1 < _NCH:
            nb = (i + 1) % _NBUF
            if i + 1 >= _NBUF:
                store(i + 1 - _NBUF, nb).wait()
            load(i + 1, nb).start()
        load(i, b).wait()
        store(i, b).start()
    for i in range(max(0, _NCH - _NBUF), _NCH):
        store(i, i % _NBUF).wait()


@jax.jit
def _copy(W):
    mesh = plsc.VectorSubcoreMesh(core_axis_name="c", subcore_axis_name="s")
    return pl.kernel(
        _copy_body,
        out_type=jax.ShapeDtypeStruct((_ROWS, _D), jnp.float32),
        mesh=mesh,
        scratch_types=[
            pltpu.VMEM((_NBUF, _C, _D), jnp.float32),
            pltpu.SemaphoreType.DMA((_NBUF,)),
            pltpu.SemaphoreType.DMA((_NBUF,)),
        ],
    )(W)


def kernel(x, W):
    del x
    return _copy(W)[None]


# speedup vs baseline: 1.0075x; 1.0056x over previous
import jax
import jax.numpy as jnp
from jax import lax
from jax.experimental import pallas as pl
from jax.experimental.pallas import tpu as pltpu, tpu_sc as plsc

_ROWS = 8192
_D = 1024
_NC = 2
_NS = 16
_NW = _NC * _NS
_RPW = _ROWS // _NW
_C = 32
_NBUF = 2
_NCH = _RPW // _C


def _copy_body(W_hbm, out_hbm, buf, lsem, ssem):
    wid = lax.axis_index("s") * _NC + lax.axis_index("c")
    base = wid * _RPW

    def load(i, b):
        return pltpu.make_async_copy(
            W_hbm.at[pl.ds(base + i * _C, _C)], buf.at[b], lsem.at[b])

    def store(i, b):
        return pltpu.make_async_copy(
            buf.at[b], out_hbm.at[pl.ds(base + i * _C, _C)], ssem.at[b])

    load(0, 0).start()
    for i in range(_NCH):
        b = i % _NBUF
        if i + 1 < _NCH:
            nb = (i + 1) % _NBUF
            if i + 1 >= _NBUF:
                store(i + 1 - _NBUF, nb).wait()
            load(i + 1, nb).start()
        load(i, b).wait()
        store(i, b).start()
    for i in range(max(0, _NCH - _NBUF), _NCH):
        store(i, i % _NBUF).wait()


@jax.jit
def _copy(W):
    mesh = plsc.VectorSubcoreMesh(core_axis_name="c", subcore_axis_name="s")
    return pl.kernel(
        _copy_body,
        out_type=jax.ShapeDtypeStruct((_ROWS, _D), jnp.float32),
        mesh=mesh,
        compiler_params=pltpu.CompilerParams(use_tc_tiling_on_sc=True),
        scratch_types=[
            pltpu.VMEM((_NBUF, _C, _D), jnp.float32),
            pltpu.SemaphoreType.DMA((_NBUF,)),
            pltpu.SemaphoreType.DMA((_NBUF,)),
        ],
    )(W)


def kernel(x, W):
    del x
    return _copy(W)[None]
